# baseline (speedup 1.0000x reference)
import dataclasses
import functools

import jax
import jax.numpy as jnp
from jax import lax
from jax.experimental import pallas as pl
from jax.experimental.pallas import tpu as pltpu
from jax.experimental.pallas import tpu_sc as plsc

N = 10000
NB = 25
RB = N // NB
CW = 512
NPAD = 10240
KNN = 8
DH = 128
DM = 32
NEG = float("-inf")
NB1 = 12
NB2 = NB - NB1
SCL = 16
NSC = 32

PREC = None


def _proj_body(x_rows, hwTp, hb2, o1wT, h_o, xo1_o):
    xr = x_rows[...]
    h_o[...] = jnp.dot(xr, hwTp[...], precision=PREC) + hb2[...]
    xo1_o[...] = jnp.dot(xr, o1wT[...], precision=PREC)


def _topk_body(cs0_ref, nch_ref, x_rows, x_all, bcol, ball, swT, sb2,
               w_o, idx_o, tv, ti):
    r = pl.program_id(0)
    xr = x_rows[...]
    s_row = jnp.dot(xr, swT[...], precision=PREC) + sb2[...]
    snr = jnp.sum(s_row * s_row, axis=1, keepdims=True)
    tv[...] = jnp.full((RB, 16), NEG, jnp.float32)
    ti[...] = jnp.zeros((RB, 16), jnp.float32)
    bcol_v = bcol[...]
    ones14 = jnp.ones((1, 4), jnp.float32)
    inf = float("inf")
    cs0 = cs0_ref[r]
    nch = nch_ref[r]

    def chunk(k, carry):
        cs = pl.multiple_of(cs0 + k * CW, CW)
        xc = x_all[pl.ds(cs, CW), :]
        sc = jnp.dot(xc, swT[...], precision=PREC) + sb2[...]
        snc = lax.dot_general(ones14, sc * sc, (((1,), (1,)), ((), ())),
                              precision=jax.lax.Precision.HIGHEST)
        cross = lax.dot_general(s_row, sc, (((1,), (1,)), ((), ())),
                                precision=PREC)
        d = jnp.maximum((snr + snc) - 2.0 * cross, 0.0)
        bc = ball[0:1, pl.ds(cs, CW)]
        dm = jnp.where(bcol_v != bc, NEG, -d)
        colid = (lax.broadcasted_iota(jnp.int32, (1, CW), 1).astype(jnp.float32)
                 + cs.astype(jnp.float32))
        for t in range(KNN):
            m = jnp.max(dm, axis=1, keepdims=True)
            am = jnp.min(jnp.where(dm == m, colid, inf), axis=1,
                         keepdims=True)
            dm = jnp.where(colid == am, NEG, dm)
            tv[:, 8 + t:9 + t] = m
            ti[:, 8 + t:9 + t] = am
        cv = tv[...]
        ci = ti[...]
        for t in range(KNN):
            m = jnp.max(cv, axis=1, keepdims=True)
            am = jnp.min(jnp.where(cv == m, ci, inf), axis=1, keepdims=True)
            cv = jnp.where(ci == am, NEG, cv)
            tv[:, t:t + 1] = m
            ti[:, t:t + 1] = am
        return carry

    lax.fori_loop(0, nch, chunk, 0)
    res_v = tv[:, 0:KNN]
    res_i = ti[:, 0:KNN]
    w_o[...] = jnp.exp(10.0 * res_v)
    idx_o[...] = jnp.where(res_v == NEG, 0.0, res_i).astype(jnp.int32)


def _head_body(agg, xo1, o2wmT, o2wxT, o2b2, w1T, b12, w2T, b22,
               w3T, b32, out_o):
    a = agg[...]
    mean = a[:, 0:DM] * (1.0 / KNN)
    mx = a[:, DM:2 * DM]
    emb = (xo1[...] + jnp.dot(mean, o2wmT[...], precision=PREC)
           + jnp.dot(mx, o2wxT[...], precision=PREC) + o2b2[...])
    h1 = jnp.dot(emb, w1T[...], precision=PREC) + b12[...]
    h1 = jnp.where(h1 > 0, h1, jnp.exp(h1) - 1.0)
    h2 = jnp.dot(h1, w2T[...], precision=PREC) + b22[...]
    h2 = jnp.where(h2 > 0, h2, jnp.exp(h2) - 1.0)
    out_o[...] = jnp.dot(h2, w3T[...], precision=PREC) + b32[...]


def _sc_combine(table, idx_flat, w_flat, nn):
    npw = nn // NSC
    bpw = npw * KNN
    chn = npw // 2
    chr_ = chn * KNN

    def body(table_hbm, idx_hbm, w_hbm, out_hbm, idx_v, w_v, rows_v,
             out_v, sem):
        wid = lax.axis_index("s") * 2 + lax.axis_index("c")
        base = wid * bpw
        nbase = wid * npw
        pltpu.sync_copy(idx_hbm.at[pl.ds(base, bpw)], idx_v)
        pltpu.sync_copy(w_hbm.at[pl.ds(base, bpw)], w_v)
        ninf = jnp.full((SCL,), float("-inf"), jnp.float32)

        @pl.loop(0, 2)
        def _(c):
            off = c * chr_
            pltpu.async_copy(table_hbm.at[idx_v.at[pl.ds(off, chr_)]],
                             rows_v, sem).wait()

            @pl.loop(0, chn)
            def _(n):
                acc0 = jnp.zeros((SCL,), jnp.float32)
                acc1 = jnp.zeros((SCL,), jnp.float32)
                m0 = ninf
                m1 = ninf
                for k in range(KNN):
                    rk = n * KNN + k
                    wv = plsc.load_gather(w_v, [jnp.full((SCL,), off + rk,
                                                         jnp.int32)])
                    t0 = wv * rows_v[rk, pl.ds(0, SCL)]
                    t1 = wv * rows_v[rk, pl.ds(SCL, SCL)]
                    acc0 = acc0 + t0
                    acc1 = acc1 + t1
                    m0 = jnp.maximum(m0, t0)
                    m1 = jnp.maximum(m1, t1)
                out_v[n, pl.ds(0, SCL)] = acc0
                out_v[n, pl.ds(SCL, SCL)] = acc1
                out_v[n, pl.ds(2 * SCL, SCL)] = m0
                out_v[n, pl.ds(3 * SCL, SCL)] = m1

            pltpu.sync_copy(out_v, out_hbm.at[pl.ds(nbase + c * chn, chn)])

    mesh = plsc.VectorSubcoreMesh(core_axis_name="c", subcore_axis_name="s")
    cp = pltpu.CompilerParams()
    if "needs_layout_passes" in pltpu.CompilerParams.__dataclass_fields__:
        cp = dataclasses.replace(cp, needs_layout_passes=False)
    gk = pl.kernel(
        body,
        mesh=mesh,
        compiler_params=cp,
        out_type=jax.ShapeDtypeStruct((nn, DH), jnp.float32),
        scratch_types=[
            pltpu.VMEM((bpw,), jnp.int32),
            pltpu.VMEM((bpw,), jnp.float32),
            pltpu.VMEM((chr_, DH), jnp.float32),
            pltpu.VMEM((chn, DH), jnp.float32),
            pltpu.SemaphoreType.DMA,
        ],
    )
    return gk(table, idx_flat, w_flat)


def kernel(x, batch, c0_sw, c0_sb, c0_hw, c0_hb, c0_o1w, c0_o2w, c0_o2b,
           c1_sw, c1_sb, c1_hw, c1_hb, c1_o1w, c1_o2w, c1_o2b,
           w1, b1, w2, b2, w3, b3):
    f32 = jnp.float32
    batch = batch.astype(jnp.int32)
    x_pad = jnp.zeros((NPAD, 34), f32).at[:N].set(x)
    ball = jnp.full((1, NPAD), -1, jnp.int32).at[0, :N].set(batch)
    bcol = batch[:, None]
    starts = jnp.searchsorted(batch, jnp.arange(33, dtype=jnp.int32)
                              ).astype(jnp.int32)
    b2d = batch.reshape(NB, RB)
    lo = starts[b2d[:, 0]]
    hi = starts[b2d[:, -1] + 1]
    cs0 = (lo // CW) * CW
    nch = (hi - cs0 + CW - 1) // CW

    swT = c1_sw.T
    sb2 = c1_sb[None, :]
    hwTp = jnp.zeros((34, DH), f32).at[:, :22].set(c1_hw.T)
    hb2 = jnp.zeros((1, DH), f32).at[0, :22].set(c1_hb)
    o1wT = c1_o1w.T
    o2wmT = jnp.zeros((DM, 34), f32).at[:22].set(c1_o2w[:, :22].T)
    o2wxT = jnp.zeros((DM, 34), f32).at[:22].set(c1_o2w[:, 22:].T)

    h_tab, xo1 = pl.pallas_call(
        _proj_body,
        grid=(5,),
        in_specs=[
            pl.BlockSpec((N // 5, 34), lambda r: (r, 0)),
            pl.BlockSpec((34, DH), lambda r: (0, 0)),
            pl.BlockSpec((1, DH), lambda r: (0, 0)),
            pl.BlockSpec((34, 34), lambda r: (0, 0)),
        ],
        out_specs=[
            pl.BlockSpec((N // 5, DH), lambda r: (r, 0)),
            pl.BlockSpec((N // 5, 34), lambda r: (r, 0)),
        ],
        out_shape=[
            jax.ShapeDtypeStruct((N, DH), f32),
            jax.ShapeDtypeStruct((N, 34), f32),
        ],
    )(x, hwTp, hb2, o1wT)

    head_w = (o2wmT, o2wxT, c1_o2b[None, :], w1.T, b1[None, :], w2.T,
              b2[None, :], w3.T, b3[None, :])
    head_specs = [
        pl.BlockSpec((DM, 34), lambda r: (0, 0)),
        pl.BlockSpec((DM, 34), lambda r: (0, 0)),
        pl.BlockSpec((1, 34), lambda r: (0, 0)),
        pl.BlockSpec((34, 126), lambda r: (0, 0)),
        pl.BlockSpec((1, 126), lambda r: (0, 0)),
        pl.BlockSpec((126, 126), lambda r: (0, 0)),
        pl.BlockSpec((1, 126), lambda r: (0, 0)),
        pl.BlockSpec((126, 6), lambda r: (0, 0)),
        pl.BlockSpec((1, 6), lambda r: (0, 0)),
    ]

    def topk_half(r0, nb):
        grid_spec = pltpu.PrefetchScalarGridSpec(
            num_scalar_prefetch=2,
            grid=(nb,),
            in_specs=[
                pl.BlockSpec((RB, 34), lambda r, *_: (r + r0, 0)),
                pl.BlockSpec((NPAD, 34), lambda r, *_: (0, 0)),
                pl.BlockSpec((RB, 1), lambda r, *_: (r + r0, 0)),
                pl.BlockSpec((1, NPAD), lambda r, *_: (0, 0)),
                pl.BlockSpec((34, 4), lambda r, *_: (0, 0)),
                pl.BlockSpec((1, 4), lambda r, *_: (0, 0)),
            ],
            out_specs=[
                pl.BlockSpec((RB, KNN), lambda r, *_: (r, 0)),
                pl.BlockSpec((RB, KNN), lambda r, *_: (r, 0)),
            ],
            scratch_shapes=[
                pltpu.VMEM((RB, 16), f32),
                pltpu.VMEM((RB, 16), f32),
            ],
        )
        return pl.pallas_call(
            _topk_body,
            grid_spec=grid_spec,
            out_shape=[
                jax.ShapeDtypeStruct((nb * RB, KNN), f32),
                jax.ShapeDtypeStruct((nb * RB, KNN), jnp.int32),
            ],
        )(cs0[r0:r0 + nb], nch[r0:r0 + nb], x, x_pad, bcol, ball, swT, sb2)

    def head_half(agg, r0, nb):
        return pl.pallas_call(
            _head_body,
            grid=(nb,),
            in_specs=[
                pl.BlockSpec((RB, DH), lambda r: (r, 0)),
                pl.BlockSpec((RB, 34), lambda r: (r + r0, 0)),
            ] + head_specs,
            out_specs=pl.BlockSpec((RB, 6), lambda r: (r, 0)),
            out_shape=jax.ShapeDtypeStruct((nb * RB, 6), f32),
        )(agg, xo1, *head_w)

    outs = []
    aggs = []
    for r0, nb in ((0, NB1), (NB1, NB2)):
        n_half = nb * RB
        nn = -(-n_half // 512) * 512
        wout, idx = topk_half(r0, nb)
        idx_flat = jnp.zeros((nn * KNN,), jnp.int32).at[:n_half * KNN].set(
            idx.reshape(n_half * KNN))
        w_flat = jnp.zeros((nn * KNN,), f32).at[:n_half * KNN].set(
            wout.reshape(n_half * KNN))
        aggs.append((_sc_combine(h_tab, idx_flat, w_flat, nn), r0, nb))
    for agg, r0, nb in aggs:
        outs.append(head_half(agg, r0, nb))
    return jnp.concatenate(outs, axis=0)


# speedup vs baseline: 1.1191x; 1.1191x over previous
import dataclasses
import functools

import jax
import jax.numpy as jnp
from jax import lax
from jax.experimental import pallas as pl
from jax.experimental.pallas import tpu as pltpu
from jax.experimental.pallas import tpu_sc as plsc

N = 10000
NB = 25
RB = N // NB
CW = 512
NPAD = 10240
KNN = 8
DH = 128
DM = 32
NEG = float("-inf")

PREC = None


def _topk_body(cs0_ref, nch_ref, x_rows, x_all, bcol, ball, swT, sb2, hwTp,
               hb2, o1wT, w_o, idx_o, h_o, xo1_o, tv, ti):
    r = pl.program_id(0)
    xr = x_rows[...]
    s_row = jnp.dot(xr, swT[...], precision=PREC) + sb2[...]
    snr = jnp.sum(s_row * s_row, axis=1, keepdims=True)
    h_o[...] = jnp.dot(xr, hwTp[...], precision=PREC) + hb2[...]
    xo1_o[...] = jnp.dot(xr, o1wT[...], precision=PREC)
    tv[...] = jnp.full((RB, 16), NEG, jnp.float32)
    ti[...] = jnp.zeros((RB, 16), jnp.float32)
    bcol_v = bcol[...]
    ones14 = jnp.ones((1, 4), jnp.float32)
    inf = float("inf")
    cs0 = cs0_ref[r]
    nch = nch_ref[r]

    def chunk(k, carry):
        cs = pl.multiple_of(cs0 + k * CW, CW)
        xc = x_all[pl.ds(cs, CW), :]
        sc = jnp.dot(xc, swT[...], precision=PREC) + sb2[...]
        snc = lax.dot_general(ones14, sc * sc, (((1,), (1,)), ((), ())),
                              precision=jax.lax.Precision.HIGHEST)
        cross = lax.dot_general(s_row, sc, (((1,), (1,)), ((), ())),
                                precision=PREC)
        d = jnp.maximum((snr + snc) - 2.0 * cross, 0.0)
        bc = ball[0:1, pl.ds(cs, CW)]
        dm = jnp.where(bcol_v != bc, NEG, -d)
        colid = (lax.broadcasted_iota(jnp.int32, (1, CW), 1).astype(jnp.float32)
                 + cs.astype(jnp.float32))
        for t in range(KNN):
            m = jnp.max(dm, axis=1, keepdims=True)
            am = jnp.min(jnp.where(dm == m, colid, inf), axis=1,
                         keepdims=True)
            dm = jnp.where(colid == am, NEG, dm)
            tv[:, 8 + t:9 + t] = m
            ti[:, 8 + t:9 + t] = am
        cv = tv[...]
        ci = ti[...]
        for t in range(KNN):
            m = jnp.max(cv, axis=1, keepdims=True)
            am = jnp.min(jnp.where(cv == m, ci, inf), axis=1, keepdims=True)
            cv = jnp.where(ci == am, NEG, cv)
            tv[:, t:t + 1] = m
            ti[:, t:t + 1] = am
        return carry

    lax.fori_loop(0, nch, chunk, 0)
    res_v = tv[:, 0:KNN]
    res_i = ti[:, 0:KNN]
    w_o[...] = jnp.exp(10.0 * res_v)
    idx_o[...] = jnp.where(res_v == NEG, 0.0, res_i).astype(jnp.int32)


def _head_body(agg, xo1, o2wmT, o2wxT, o2b2, w1T, b12, w2T, b22,
               w3T, b32, out_o):
    a = agg[...]
    mean = a[:, 0:DM] * (1.0 / KNN)
    mx = a[:, DM:2 * DM]
    emb = (xo1[...] + jnp.dot(mean, o2wmT[...], precision=PREC)
           + jnp.dot(mx, o2wxT[...], precision=PREC) + o2b2[...])
    h1 = jnp.dot(emb, w1T[...], precision=PREC) + b12[...]
    h1 = jnp.where(h1 > 0, h1, jnp.exp(h1) - 1.0)
    h2 = jnp.dot(h1, w2T[...], precision=PREC) + b22[...]
    h2 = jnp.where(h2 > 0, h2, jnp.exp(h2) - 1.0)
    out_o[...] = jnp.dot(h2, w3T[...], precision=PREC) + b32[...]


_SC_B = 81920
_SC_BPW = _SC_B // 32
_SC_CH = 640
_SC_NCH = _SC_BPW // _SC_CH
_SC_NPW = _SC_BPW // KNN
_SC_CHN = _SC_CH // KNN
NAGG = 10240
SCL = 16


def _sc_combine_body(table_hbm, idx_hbm, w_hbm, out_hbm, idx_v, w_v, rows_v,
                     out_v, sem):
    wid = lax.axis_index("s") * 2 + lax.axis_index("c")
    base = wid * _SC_BPW
    nbase = wid * _SC_NPW
    pltpu.sync_copy(idx_hbm.at[pl.ds(base, _SC_BPW)], idx_v)
    pltpu.sync_copy(w_hbm.at[pl.ds(base, _SC_BPW)], w_v)
    ninf = jnp.full((SCL,), float("-inf"), jnp.float32)

    @pl.loop(0, _SC_NCH)
    def _(c):
        off = c * _SC_CH
        pltpu.async_copy(table_hbm.at[idx_v.at[pl.ds(off, _SC_CH)]],
                         rows_v, sem).wait()

        @pl.loop(0, _SC_CHN)
        def _(n):
            acc0 = jnp.zeros((SCL,), jnp.float32)
            acc1 = jnp.zeros((SCL,), jnp.float32)
            m0 = ninf
            m1 = ninf
            for k in range(KNN):
                rk = n * KNN + k
                wv = plsc.load_gather(w_v, [jnp.full((SCL,), off + rk,
                                                     jnp.int32)])
                t0 = wv * rows_v[rk, pl.ds(0, SCL)]
                t1 = wv * rows_v[rk, pl.ds(SCL, SCL)]
                acc0 = acc0 + t0
                acc1 = acc1 + t1
                m0 = jnp.maximum(m0, t0)
                m1 = jnp.maximum(m1, t1)
            out_v[n, pl.ds(0, SCL)] = acc0
            out_v[n, pl.ds(SCL, SCL)] = acc1
            out_v[n, pl.ds(2 * SCL, SCL)] = m0
            out_v[n, pl.ds(3 * SCL, SCL)] = m1

        pltpu.sync_copy(out_v, out_hbm.at[pl.ds(nbase + c * _SC_CHN,
                                                _SC_CHN)])


def _sc_combine(table, idx_flat, w_flat):
    mesh = plsc.VectorSubcoreMesh(core_axis_name="c", subcore_axis_name="s")
    cp = pltpu.CompilerParams()
    if "needs_layout_passes" in pltpu.CompilerParams.__dataclass_fields__:
        cp = dataclasses.replace(cp, needs_layout_passes=False)
    gk = pl.kernel(
        _sc_combine_body,
        mesh=mesh,
        compiler_params=cp,
        out_type=jax.ShapeDtypeStruct((NAGG, DH), jnp.float32),
        scratch_types=[
            pltpu.VMEM((_SC_BPW,), jnp.int32),
            pltpu.VMEM((_SC_BPW,), jnp.float32),
            pltpu.VMEM((_SC_CH, DH), jnp.float32),
            pltpu.VMEM((_SC_CHN, DH), jnp.float32),
            pltpu.SemaphoreType.DMA,
        ],
    )
    return gk(table, idx_flat, w_flat)


def kernel(x, batch, c0_sw, c0_sb, c0_hw, c0_hb, c0_o1w, c0_o2w, c0_o2b,
           c1_sw, c1_sb, c1_hw, c1_hb, c1_o1w, c1_o2w, c1_o2b,
           w1, b1, w2, b2, w3, b3):
    f32 = jnp.float32
    batch = batch.astype(jnp.int32)
    x_pad = jnp.zeros((NPAD, 34), f32).at[:N].set(x)
    ball = jnp.full((1, NPAD), -1, jnp.int32).at[0, :N].set(batch)
    bcol = batch[:, None]
    starts = jnp.searchsorted(batch, jnp.arange(33, dtype=jnp.int32)
                              ).astype(jnp.int32)
    b2d = batch.reshape(NB, RB)
    lo = starts[b2d[:, 0]]
    hi = starts[b2d[:, -1] + 1]
    cs0 = (lo // CW) * CW
    nch = (hi - cs0 + CW - 1) // CW

    swT = c1_sw.T
    sb2 = c1_sb[None, :]
    hwTp = jnp.zeros((34, DH), f32).at[:, :22].set(c1_hw.T)
    hb2 = jnp.zeros((1, DH), f32).at[0, :22].set(c1_hb)
    o1wT = c1_o1w.T
    o2wmT = jnp.zeros((DM, 34), f32).at[:22].set(c1_o2w[:, :22].T)
    o2wxT = jnp.zeros((DM, 34), f32).at[:22].set(c1_o2w[:, 22:].T)

    grid_spec = pltpu.PrefetchScalarGridSpec(
        num_scalar_prefetch=2,
        grid=(NB,),
        in_specs=[
            pl.BlockSpec((RB, 34), lambda r, *_: (r, 0)),
            pl.BlockSpec((NPAD, 34), lambda r, *_: (0, 0)),
            pl.BlockSpec((RB, 1), lambda r, *_: (r, 0)),
            pl.BlockSpec((1, NPAD), lambda r, *_: (0, 0)),
            pl.BlockSpec((34, 4), lambda r, *_: (0, 0)),
            pl.BlockSpec((1, 4), lambda r, *_: (0, 0)),
            pl.BlockSpec((34, DH), lambda r, *_: (0, 0)),
            pl.BlockSpec((1, DH), lambda r, *_: (0, 0)),
            pl.BlockSpec((34, 34), lambda r, *_: (0, 0)),
        ],
        out_specs=[
            pl.BlockSpec((RB, KNN), lambda r, *_: (r, 0)),
            pl.BlockSpec((RB, KNN), lambda r, *_: (r, 0)),
            pl.BlockSpec((RB, DH), lambda r, *_: (r, 0)),
            pl.BlockSpec((RB, 34), lambda r, *_: (r, 0)),
        ],
        scratch_shapes=[
            pltpu.VMEM((RB, 16), f32),
            pltpu.VMEM((RB, 16), f32),
        ],
    )
    wout, idx, h_tab, xo1 = pl.pallas_call(
        _topk_body,
        grid_spec=grid_spec,
        out_shape=[
            jax.ShapeDtypeStruct((N, KNN), f32),
            jax.ShapeDtypeStruct((N, KNN), jnp.int32),
            jax.ShapeDtypeStruct((N, DH), f32),
            jax.ShapeDtypeStruct((N, 34), f32),
        ],
    )(cs0, nch, x, x_pad, bcol, ball, swT, sb2, hwTp, hb2, o1wT)

    idx_flat = jnp.zeros((_SC_B,), jnp.int32).at[:N * KNN].set(
        idx.reshape(N * KNN))
    w_flat = jnp.zeros((_SC_B,), f32).at[:N * KNN].set(
        wout.reshape(N * KNN))
    agg = _sc_combine(h_tab, idx_flat, w_flat)

    out = pl.pallas_call(
        _head_body,
        grid=(NB,),
        in_specs=[
            pl.BlockSpec((RB, DH), lambda r: (r, 0)),
            pl.BlockSpec((RB, 34), lambda r: (r, 0)),
            pl.BlockSpec((DM, 34), lambda r: (0, 0)),
            pl.BlockSpec((DM, 34), lambda r: (0, 0)),
            pl.BlockSpec((1, 34), lambda r: (0, 0)),
            pl.BlockSpec((34, 126), lambda r: (0, 0)),
            pl.BlockSpec((1, 126), lambda r: (0, 0)),
            pl.BlockSpec((126, 126), lambda r: (0, 0)),
            pl.BlockSpec((1, 126), lambda r: (0, 0)),
            pl.BlockSpec((126, 6), lambda r: (0, 0)),
            pl.BlockSpec((1, 6), lambda r: (0, 0)),
        ],
        out_specs=pl.BlockSpec((RB, 6), lambda r: (r, 0)),
        out_shape=jax.ShapeDtypeStruct((N, 6), f32),
    )(agg, xo1, o2wmT, o2wxT,
      c1_o2b[None, :], w1.T, b1[None, :], w2.T, b2[None, :],
      w3.T, b3[None, :])
    return out


# speedup vs baseline: 1.1312x; 1.0108x over previous
import dataclasses
import functools

import jax
import jax.numpy as jnp
from jax import lax
from jax.experimental import pallas as pl
from jax.experimental.pallas import tpu as pltpu
from jax.experimental.pallas import tpu_sc as plsc

N = 10000
NB = 25
RB = N // NB
CW = 512
NPAD = 10240
KNN = 8
DH = 128
DM = 32
NEG = float("-inf")

PREC = None


def _topk_body(cs0_ref, nch_ref, x_rows, x_all, bcol, ball, swT, sb2, hwTp,
               hb2, o1wT, w_o, idx_o, h_o, xo1_o, tv, ti):
    r = pl.program_id(0)
    xr = x_rows[...]
    s_row = jnp.dot(xr, swT[...], precision=PREC) + sb2[...]
    snr = jnp.sum(s_row * s_row, axis=1, keepdims=True)
    h_o[...] = jnp.dot(xr, hwTp[...], precision=PREC) + hb2[...]
    xo1_o[...] = jnp.dot(xr, o1wT[...], precision=PREC)
    tv[...] = jnp.full((RB, 16), NEG, jnp.float32)
    ti[...] = jnp.zeros((RB, 16), jnp.float32)
    bcol_v = bcol[...]
    ones14 = jnp.ones((1, 4), jnp.float32)
    inf = float("inf")
    cs0 = cs0_ref[r]
    nch = nch_ref[r]

    def chunk(k, carry):
        cs = pl.multiple_of(cs0 + k * CW, CW)
        xc = x_all[pl.ds(cs, CW), :]
        sc = jnp.dot(xc, swT[...], precision=PREC) + sb2[...]
        snc = lax.dot_general(ones14, sc * sc, (((1,), (1,)), ((), ())),
                              precision=jax.lax.Precision.HIGHEST)
        cross = lax.dot_general(s_row, sc, (((1,), (1,)), ((), ())),
                                precision=PREC)
        d = jnp.maximum((snr + snc) - 2.0 * cross, 0.0)
        bc = ball[0:1, pl.ds(cs, CW)]
        dm = jnp.where(bcol_v != bc, NEG, -d)
        colid = (lax.broadcasted_iota(jnp.int32, (1, CW), 1).astype(jnp.float32)
                 + cs.astype(jnp.float32))
        for t in range(KNN):
            m = jnp.max(dm, axis=1, keepdims=True)
            am = jnp.min(jnp.where(dm == m, colid, inf), axis=1,
                         keepdims=True)
            dm = jnp.where(colid == am, NEG, dm)
            tv[:, 8 + t:9 + t] = m
            ti[:, 8 + t:9 + t] = am
        cv = tv[...]
        ci = ti[...]
        for t in range(KNN):
            m = jnp.max(cv, axis=1, keepdims=True)
            am = jnp.min(jnp.where(cv == m, ci, inf), axis=1, keepdims=True)
            cv = jnp.where(ci == am, NEG, cv)
            tv[:, t:t + 1] = m
            ti[:, t:t + 1] = am
        return carry

    lax.fori_loop(0, nch, chunk, 0)
    res_v = tv[:, 0:KNN]
    res_i = ti[:, 0:KNN]
    w_o[...] = jnp.exp(10.0 * res_v)
    idx_o[...] = jnp.where(res_v == NEG, 0.0, res_i).astype(jnp.int32)


def _head_body(agg, xo1, o2wmT, o2wxT, o2b2, w1T, b12, w2T, b22,
               w3T, b32, out_o):
    a = agg[...]
    mean = a[:, 0:DM] * (1.0 / KNN)
    mx = a[:, DM:2 * DM]
    emb = (xo1[...] + jnp.dot(mean, o2wmT[...], precision=PREC)
           + jnp.dot(mx, o2wxT[...], precision=PREC) + o2b2[...])
    h1 = jnp.dot(emb, w1T[...], precision=PREC) + b12[...]
    h1 = jnp.where(h1 > 0, h1, jnp.exp(h1) - 1.0)
    h2 = jnp.dot(h1, w2T[...], precision=PREC) + b22[...]
    h2 = jnp.where(h2 > 0, h2, jnp.exp(h2) - 1.0)
    out_o[...] = jnp.dot(h2, w3T[...], precision=PREC) + b32[...]


_SC_B = 81920
_SC_BPW = _SC_B // 32
_SC_CH = 320
_SC_NCH = _SC_BPW // _SC_CH
_SC_NPW = _SC_BPW // KNN
_SC_CHN = _SC_CH // KNN
NAGG = 10240
SCL = 16


def _sc_combine_body(table_hbm, idx_hbm, w_hbm, out_hbm, idx_v, w_v,
                     rows_v0, rows_v1, out_v0, out_v1, sem0, sem1):
    wid = lax.axis_index("s") * 2 + lax.axis_index("c")
    base = wid * _SC_BPW
    nbase = wid * _SC_NPW
    pltpu.sync_copy(idx_hbm.at[pl.ds(base, _SC_BPW)], idx_v)
    pltpu.sync_copy(w_hbm.at[pl.ds(base, _SC_BPW)], w_v)
    ninf = jnp.full((SCL,), float("-inf"), jnp.float32)
    bufs = ((rows_v0, out_v0, sem0), (rows_v1, out_v1, sem1))
    pending = [None, None]
    pending[0] = pltpu.async_copy(
        table_hbm.at[idx_v.at[pl.ds(0, _SC_CH)]], rows_v0, sem0)
    for c in range(_SC_NCH):
        rows_v, out_v, _ = bufs[c % 2]
        off = c * _SC_CH
        pending[c % 2].wait()
        if c + 1 < _SC_NCH:
            nrows, _, nsem = bufs[(c + 1) % 2]
            pending[(c + 1) % 2] = pltpu.async_copy(
                table_hbm.at[idx_v.at[pl.ds(off + _SC_CH, _SC_CH)]],
                nrows, nsem)

        @pl.loop(0, _SC_CHN)
        def _(n, off=off, rows_v=rows_v, out_v=out_v):
            acc0 = jnp.zeros((SCL,), jnp.float32)
            acc1 = jnp.zeros((SCL,), jnp.float32)
            m0 = ninf
            m1 = ninf
            for k in range(KNN):
                rk = n * KNN + k
                wv = plsc.load_gather(w_v, [jnp.full((SCL,), off + rk,
                                                     jnp.int32)])
                t0 = wv * rows_v[rk, pl.ds(0, SCL)]
                t1 = wv * rows_v[rk, pl.ds(SCL, SCL)]
                acc0 = acc0 + t0
                acc1 = acc1 + t1
                m0 = jnp.maximum(m0, t0)
                m1 = jnp.maximum(m1, t1)
            out_v[n, pl.ds(0, SCL)] = acc0
            out_v[n, pl.ds(SCL, SCL)] = acc1
            out_v[n, pl.ds(2 * SCL, SCL)] = m0
            out_v[n, pl.ds(3 * SCL, SCL)] = m1

        pltpu.sync_copy(out_v, out_hbm.at[pl.ds(nbase + c * _SC_CHN,
                                                _SC_CHN)])


def _sc_combine(table, idx_flat, w_flat):
    mesh = plsc.VectorSubcoreMesh(core_axis_name="c", subcore_axis_name="s")
    cp = pltpu.CompilerParams()
    if "needs_layout_passes" in pltpu.CompilerParams.__dataclass_fields__:
        cp = dataclasses.replace(cp, needs_layout_passes=False)
    gk = pl.kernel(
        _sc_combine_body,
        mesh=mesh,
        compiler_params=cp,
        out_type=jax.ShapeDtypeStruct((NAGG, DH), jnp.float32),
        scratch_types=[
            pltpu.VMEM((_SC_BPW,), jnp.int32),
            pltpu.VMEM((_SC_BPW,), jnp.float32),
            pltpu.VMEM((_SC_CH, DH), jnp.float32),
            pltpu.VMEM((_SC_CH, DH), jnp.float32),
            pltpu.VMEM((_SC_CHN, DH), jnp.float32),
            pltpu.VMEM((_SC_CHN, DH), jnp.float32),
            pltpu.SemaphoreType.DMA,
            pltpu.SemaphoreType.DMA,
        ],
    )
    return gk(table, idx_flat, w_flat)


def kernel(x, batch, c0_sw, c0_sb, c0_hw, c0_hb, c0_o1w, c0_o2w, c0_o2b,
           c1_sw, c1_sb, c1_hw, c1_hb, c1_o1w, c1_o2w, c1_o2b,
           w1, b1, w2, b2, w3, b3):
    f32 = jnp.float32
    batch = batch.astype(jnp.int32)
    x_pad = jnp.zeros((NPAD, 34), f32).at[:N].set(x)
    ball = jnp.full((1, NPAD), -1, jnp.int32).at[0, :N].set(batch)
    bcol = batch[:, None]
    starts = jnp.searchsorted(batch, jnp.arange(33, dtype=jnp.int32)
                              ).astype(jnp.int32)
    b2d = batch.reshape(NB, RB)
    lo = starts[b2d[:, 0]]
    hi = starts[b2d[:, -1] + 1]
    cs0 = (lo // CW) * CW
    nch = (hi - cs0 + CW - 1) // CW

    swT = c1_sw.T
    sb2 = c1_sb[None, :]
    hwTp = jnp.zeros((34, DH), f32).at[:, :22].set(c1_hw.T)
    hb2 = jnp.zeros((1, DH), f32).at[0, :22].set(c1_hb)
    o1wT = c1_o1w.T
    o2wmT = jnp.zeros((DM, 34), f32).at[:22].set(c1_o2w[:, :22].T)
    o2wxT = jnp.zeros((DM, 34), f32).at[:22].set(c1_o2w[:, 22:].T)

    grid_spec = pltpu.PrefetchScalarGridSpec(
        num_scalar_prefetch=2,
        grid=(NB,),
        in_specs=[
            pl.BlockSpec((RB, 34), lambda r, *_: (r, 0)),
            pl.BlockSpec((NPAD, 34), lambda r, *_: (0, 0)),
            pl.BlockSpec((RB, 1), lambda r, *_: (r, 0)),
            pl.BlockSpec((1, NPAD), lambda r, *_: (0, 0)),
            pl.BlockSpec((34, 4), lambda r, *_: (0, 0)),
            pl.BlockSpec((1, 4), lambda r, *_: (0, 0)),
            pl.BlockSpec((34, DH), lambda r, *_: (0, 0)),
            pl.BlockSpec((1, DH), lambda r, *_: (0, 0)),
            pl.BlockSpec((34, 34), lambda r, *_: (0, 0)),
        ],
        out_specs=[
            pl.BlockSpec((RB, KNN), lambda r, *_: (r, 0)),
            pl.BlockSpec((RB, KNN), lambda r, *_: (r, 0)),
            pl.BlockSpec((RB, DH), lambda r, *_: (r, 0)),
            pl.BlockSpec((RB, 34), lambda r, *_: (r, 0)),
        ],
        scratch_shapes=[
            pltpu.VMEM((RB, 16), f32),
            pltpu.VMEM((RB, 16), f32),
        ],
    )
    wout, idx, h_tab, xo1 = pl.pallas_call(
        _topk_body,
        grid_spec=grid_spec,
        out_shape=[
            jax.ShapeDtypeStruct((N, KNN), f32),
            jax.ShapeDtypeStruct((N, KNN), jnp.int32),
            jax.ShapeDtypeStruct((N, DH), f32),
            jax.ShapeDtypeStruct((N, 34), f32),
        ],
    )(cs0, nch, x, x_pad, bcol, ball, swT, sb2, hwTp, hb2, o1wT)

    idx_flat = jnp.zeros((_SC_B,), jnp.int32).at[:N * KNN].set(
        idx.reshape(N * KNN))
    w_flat = jnp.zeros((_SC_B,), f32).at[:N * KNN].set(
        wout.reshape(N * KNN))
    agg = _sc_combine(h_tab, idx_flat, w_flat)

    out = pl.pallas_call(
        _head_body,
        grid=(NB,),
        in_specs=[
            pl.BlockSpec((RB, DH), lambda r: (r, 0)),
            pl.BlockSpec((RB, 34), lambda r: (r, 0)),
            pl.BlockSpec((DM, 34), lambda r: (0, 0)),
            pl.BlockSpec((DM, 34), lambda r: (0, 0)),
            pl.BlockSpec((1, 34), lambda r: (0, 0)),
            pl.BlockSpec((34, 126), lambda r: (0, 0)),
            pl.BlockSpec((1, 126), lambda r: (0, 0)),
            pl.BlockSpec((126, 126), lambda r: (0, 0)),
            pl.BlockSpec((1, 126), lambda r: (0, 0)),
            pl.BlockSpec((126, 6), lambda r: (0, 0)),
            pl.BlockSpec((1, 6), lambda r: (0, 0)),
        ],
        out_specs=pl.BlockSpec((RB, 6), lambda r: (r, 0)),
        out_shape=jax.ShapeDtypeStruct((N, 6), f32),
    )(agg, xo1, o2wmT, o2wxT,
      c1_o2b[None, :], w1.T, b1[None, :], w2.T, b2[None, :],
      w3.T, b3[None, :])
    return out
